# baseline (speedup 1.0000x reference)
import jax
import jax.numpy as jnp
from jax import lax
from jax.experimental import pallas as pl
from jax.experimental.pallas import tpu as pltpu
from jax.experimental.pallas import tpu_sc as plsc

_NC, _NS = 2, 16
_NW = _NC * _NS


def _argmin_body(x_ref, dict_ref, lane_ref, idx_ref, loss_ref):
    i = pl.program_id(0)
    x = x_ref[...]
    d = dict_ref[...]
    lane = lane_ref[...]
    k = d.shape[1]
    scores2 = jnp.dot(x + x, d, preferred_element_type=jnp.float32)
    norms = jnp.sum(d * d, axis=0, keepdims=True)
    row_norms = jnp.sum(x * x, axis=1, keepdims=True)
    dist = (row_norms + norms) - scores2
    m = jnp.min(dist, axis=1, keepdims=True)
    idxf = jnp.min(jnp.where(dist == m, lane, float(k)), axis=1, keepdims=True)
    idx_ref[...] = idxf.astype(jnp.int32)
    part = jnp.sum(m).reshape(1, 1)

    @pl.when(i == 0)
    def _init():
        loss_ref[...] = jnp.zeros_like(loss_ref)

    loss_ref[...] += part


def _sc_gather_body(idx_hbm, table_hbm, q_hbm, idx_v, rows_v, sem):
    wid = lax.axis_index("s") * _NC + lax.axis_index("c")
    chunks = idx_v.shape[0]
    bpw = rows_v.shape[0]
    pltpu.sync_copy(idx_hbm.at[pl.ds(wid * chunks, chunks)], idx_v)
    copies = [
        pltpu.async_copy(
            table_hbm.at[idx_v.at[j]],
            rows_v.at[pl.ds(j * 128, 128)],
            sem,
        )
        for j in range(chunks)
    ]
    for cp in copies:
        cp.wait()
    pltpu.sync_copy(rows_v, q_hbm.at[pl.ds(wid * bpw, bpw)])


def kernel(x, dictionary):
    beta = 0.25
    img_dims = x.shape
    d_dim, k_dim = dictionary.shape
    flat = x.reshape(-1, d_dim)
    n = flat.shape[0]
    blk = 4096
    dict_t = dictionary.T
    lane_row = jnp.arange(k_dim, dtype=jnp.float32).reshape(1, k_dim)

    idx, loss_sum = pl.pallas_call(
        _argmin_body,
        grid=(n // blk,),
        in_specs=[
            pl.BlockSpec((blk, d_dim), lambda i: (i, 0)),
            pl.BlockSpec((d_dim, k_dim), lambda i: (0, 0)),
            pl.BlockSpec((1, k_dim), lambda i: (0, 0)),
        ],
        out_specs=[
            pl.BlockSpec((blk, 1), lambda i: (i, 0)),
            pl.BlockSpec((1, 1), lambda i: (0, 0)),
        ],
        out_shape=[
            jax.ShapeDtypeStruct((n, 1), jnp.int32),
            jax.ShapeDtypeStruct((1, 1), jnp.float32),
        ],
    )(flat, dictionary, lane_row)

    bpw = n // _NW
    chunks = bpw // 128
    idx2d = idx.reshape(n // 128, 128)

    mesh = plsc.VectorSubcoreMesh(
        core_axis_name="c", subcore_axis_name="s",
        num_cores=_NC, num_subcores=_NS,
    )
    q_flat = pl.kernel(
        _sc_gather_body,
        out_type=jax.ShapeDtypeStruct((n, d_dim), jnp.float32),
        mesh=mesh,
        compiler_params=pltpu.CompilerParams(use_tc_tiling_on_sc=False),
        scratch_types=[
            pltpu.VMEM((chunks, 128), jnp.int32),
            pltpu.VMEM((bpw, d_dim), jnp.float32),
            pltpu.SemaphoreType.DMA,
        ],
    )(idx2d, dict_t)

    q = q_flat.reshape(img_dims)
    loss = (1.0 + beta) * loss_sum[0, 0] / x.size
    return q, loss


# speedup vs baseline: 1.8800x; 1.8800x over previous
import jax
import jax.numpy as jnp
from jax.experimental import pallas as pl


def _vq_body(x_ref, dict_ref, dict_t_ref, lane_ref, q_ref, loss_ref):
    i = pl.program_id(0)
    x = x_ref[...]
    d = dict_ref[...]
    dt = dict_t_ref[...]
    lane = lane_ref[...]
    k = d.shape[1]
    scores2 = jnp.dot(x + x, d, preferred_element_type=jnp.float32)
    norms = jnp.sum(d * d, axis=0, keepdims=True)
    row_norms = jnp.sum(x * x, axis=1, keepdims=True)
    dist = (row_norms + norms) - scores2
    m = jnp.min(dist, axis=1, keepdims=True)
    idx = jnp.min(jnp.where(dist == m, lane, float(k)), axis=1, keepdims=True)
    onehot = (lane == idx).astype(jnp.float32)
    q = jnp.dot(onehot, dt, preferred_element_type=jnp.float32)
    q_ref[...] = q
    part = jnp.sum(m).reshape(1, 1)

    @pl.when(i == 0)
    def _init():
        loss_ref[...] = jnp.zeros_like(loss_ref)

    loss_ref[...] += part


def kernel(x, dictionary):
    beta = 0.25
    img_dims = x.shape
    d_dim, k_dim = dictionary.shape
    flat = x.reshape(-1, d_dim)
    n = flat.shape[0]
    blk = 4096
    dict_t = dictionary.T
    lane_row = jnp.arange(k_dim, dtype=jnp.float32).reshape(1, k_dim)

    q, loss_sum = pl.pallas_call(
        _vq_body,
        grid=(n // blk,),
        in_specs=[
            pl.BlockSpec((blk, d_dim), lambda i: (i, 0)),
            pl.BlockSpec((d_dim, k_dim), lambda i: (0, 0)),
            pl.BlockSpec((k_dim, d_dim), lambda i: (0, 0)),
            pl.BlockSpec((1, k_dim), lambda i: (0, 0)),
        ],
        out_specs=[
            pl.BlockSpec((blk, d_dim), lambda i: (i, 0)),
            pl.BlockSpec((1, 1), lambda i: (0, 0)),
        ],
        out_shape=[
            jax.ShapeDtypeStruct((n, d_dim), jnp.float32),
            jax.ShapeDtypeStruct((1, 1), jnp.float32),
        ],
    )(flat, dictionary, dict_t, lane_row)

    q = q.reshape(img_dims)
    loss = (1.0 + beta) * loss_sum[0, 0] / x.size
    return q, loss
